# baseline (speedup 1.0000x reference)
import jax
import jax.numpy as jnp
from jax.experimental import pallas as pl
from jax.experimental.pallas import tpu as pltpu

_N = 10000
_INFEAT = 128
_HIDDEN = 24
_OUTFEAT = 128
_BM = 400
_BH = _BM // 2


def _body(x_ref, adj_a_ref, adj_b_ref, w1_ref, b1_ref, w2_ref, b2_ref,
          o_ref, s_ref):
    @pl.when(pl.program_id(0) == 0)
    def _():
        s_ref[...] = jnp.dot(x_ref[...], w1_ref[...],
                             preferred_element_type=jnp.float32)

    acc_a = jnp.dot(adj_a_ref[...], s_ref[...],
                    preferred_element_type=jnp.float32)
    h_a = acc_a + b1_ref[...]
    h_a = jnp.where(h_a > 0, h_a, 0.01 * h_a)
    o_ref[:_BH, :] = jnp.tanh(
        jnp.dot(h_a, w2_ref[...], preferred_element_type=jnp.float32)
        + b2_ref[...])

    acc_b = jnp.dot(adj_b_ref[...], s_ref[...],
                    preferred_element_type=jnp.float32)
    h_b = acc_b + b1_ref[...]
    h_b = jnp.where(h_b > 0, h_b, 0.01 * h_b)
    o_ref[_BH:, :] = jnp.tanh(
        jnp.dot(h_b, w2_ref[...], preferred_element_type=jnp.float32)
        + b2_ref[...])


def kernel(x, adj, W1, b1, W2, b2):
    b1r = b1.reshape(1, _HIDDEN)
    b2r = b2.reshape(1, _OUTFEAT)

    return pl.pallas_call(
        _body,
        grid=(_N // _BM,),
        in_specs=[
            pl.BlockSpec((_N, _INFEAT), lambda i: (0, 0)),
            pl.BlockSpec((_BH, _N), lambda i: (2 * i, 0)),
            pl.BlockSpec((_BH, _N), lambda i: (2 * i + 1, 0)),
            pl.BlockSpec((_INFEAT, _HIDDEN), lambda i: (0, 0)),
            pl.BlockSpec((1, _HIDDEN), lambda i: (0, 0)),
            pl.BlockSpec((_HIDDEN, _OUTFEAT), lambda i: (0, 0)),
            pl.BlockSpec((1, _OUTFEAT), lambda i: (0, 0)),
        ],
        out_specs=pl.BlockSpec((_BM, _OUTFEAT), lambda i: (i, 0)),
        out_shape=jax.ShapeDtypeStruct((_N, _OUTFEAT), jnp.float32),
        scratch_shapes=[pltpu.VMEM((_N, _HIDDEN), jnp.float32)],
    )(x, adj, adj, W1, b1r, W2, b2r)


# speedup vs baseline: 1.0769x; 1.0769x over previous
import jax
import jax.numpy as jnp
from jax.experimental import pallas as pl
from jax.experimental.pallas import tpu as pltpu

_N = 10000
_INFEAT = 128
_HIDDEN = 24
_OUTFEAT = 128
_BK = 400
_NB = _N // _BK
_NS = 3


def _body(adj_hbm, x_hbm, w1_ref, b1_ref, w2_ref, b2_ref, o_hbm,
          x_vmem, s_ref, bufs, ostg, adj_sems, x_sem, out_sems):
    xcp = pltpu.make_async_copy(x_hbm, x_vmem, x_sem)
    xcp.start()
    for k in range(_NS):
        pltpu.make_async_copy(
            adj_hbm.at[pl.ds(k * _BK, _BK), :],
            bufs.at[pl.ds(k * _BK, _BK), :],
            adj_sems.at[k]).start()
    xcp.wait()
    s_ref[...] = jnp.dot(x_vmem[...], w1_ref[...],
                         preferred_element_type=jnp.float32)

    def step(i, carry):
        slot = jax.lax.rem(i, _NS)
        oslot = jax.lax.rem(i, 2)
        pltpu.make_async_copy(
            adj_hbm.at[pl.ds(i * _BK, _BK), :],
            bufs.at[pl.ds(slot * _BK, _BK), :],
            adj_sems.at[slot]).wait()
        acc = jnp.dot(bufs[pl.ds(slot * _BK, _BK), :], s_ref[...],
                      preferred_element_type=jnp.float32)
        nxt = i + _NS

        @pl.when(nxt < _NB)
        def _():
            pltpu.make_async_copy(
                adj_hbm.at[pl.ds(nxt * _BK, _BK), :],
                bufs.at[pl.ds(slot * _BK, _BK), :],
                adj_sems.at[slot]).start()

        h = acc + b1_ref[...]
        h = jnp.where(h > 0, h, 0.01 * h)
        r = jnp.tanh(
            jnp.dot(h, w2_ref[...], preferred_element_type=jnp.float32)
            + b2_ref[...])

        @pl.when(i >= 2)
        def _():
            pltpu.make_async_copy(
                ostg.at[pl.ds(oslot * _BK, _BK), :],
                o_hbm.at[pl.ds((i - 2) * _BK, _BK), :],
                out_sems.at[oslot]).wait()

        ostg[pl.ds(oslot * _BK, _BK), :] = r
        pltpu.make_async_copy(
            ostg.at[pl.ds(oslot * _BK, _BK), :],
            o_hbm.at[pl.ds(i * _BK, _BK), :],
            out_sems.at[oslot]).start()
        return carry

    jax.lax.fori_loop(0, _NB, step, 0)
    for j in (_NB - 2, _NB - 1):
        oslot = j % 2
        pltpu.make_async_copy(
            ostg.at[pl.ds(oslot * _BK, _BK), :],
            o_hbm.at[pl.ds(j * _BK, _BK), :],
            out_sems.at[oslot]).wait()


def kernel(x, adj, W1, b1, W2, b2):
    b1r = b1.reshape(1, _HIDDEN)
    b2r = b2.reshape(1, _OUTFEAT)

    return pl.pallas_call(
        _body,
        in_specs=[
            pl.BlockSpec(memory_space=pltpu.MemorySpace.HBM),
            pl.BlockSpec(memory_space=pltpu.MemorySpace.HBM),
            pl.BlockSpec(memory_space=pltpu.MemorySpace.VMEM),
            pl.BlockSpec(memory_space=pltpu.MemorySpace.VMEM),
            pl.BlockSpec(memory_space=pltpu.MemorySpace.VMEM),
            pl.BlockSpec(memory_space=pltpu.MemorySpace.VMEM),
        ],
        out_specs=pl.BlockSpec(memory_space=pltpu.MemorySpace.HBM),
        out_shape=jax.ShapeDtypeStruct((_N, _OUTFEAT), jnp.float32),
        scratch_shapes=[
            pltpu.VMEM((_N, _INFEAT), jnp.float32),
            pltpu.VMEM((_N, _HIDDEN), jnp.float32),
            pltpu.VMEM((_NS * _BK, _N), jnp.float32),
            pltpu.VMEM((2 * _BK, _OUTFEAT), jnp.float32),
            pltpu.SemaphoreType.DMA((_NS,)),
            pltpu.SemaphoreType.DMA,
            pltpu.SemaphoreType.DMA((2,)),
        ],
        compiler_params=pltpu.CompilerParams(
            vmem_limit_bytes=64 * 1024 * 1024),
    )(adj, x, W1, b1r, W2, b2r)


# speedup vs baseline: 1.0856x; 1.0081x over previous
import jax
import jax.numpy as jnp
from jax.experimental import pallas as pl
from jax.experimental.pallas import tpu as pltpu

_N = 10000
_INFEAT = 128
_HIDDEN = 24
_OUTFEAT = 128
_BK = 400
_NB = _N // _BK
_NS = 3


def _body(adj_hbm, x_hbm, w1_ref, b1_ref, w2_ref, b2_ref, o_hbm,
          x_vmem, s_ref, bufs, ostg, adj_sems, x_sem, out_sems):
    xcp = pltpu.make_async_copy(x_hbm, x_vmem, x_sem)
    xcp.start()
    for k in range(_NS):
        pltpu.make_async_copy(
            adj_hbm.at[pl.ds(k * _BK, _BK), :],
            bufs.at[pl.ds(k * _BK, _BK), :],
            adj_sems.at[k]).start()
    xcp.wait()
    s_ref[...] = jnp.dot(x_vmem[...], w1_ref[...],
                         preferred_element_type=jnp.float32)

    for i in range(_NB):
        slot = i % _NS
        oslot = i % 2
        pltpu.make_async_copy(
            adj_hbm.at[pl.ds(i * _BK, _BK), :],
            bufs.at[pl.ds(slot * _BK, _BK), :],
            adj_sems.at[slot]).wait()
        acc = jnp.dot(bufs[pl.ds(slot * _BK, _BK), :], s_ref[...],
                      preferred_element_type=jnp.float32)
        nxt = i + _NS
        if nxt < _NB:
            pltpu.make_async_copy(
                adj_hbm.at[pl.ds(nxt * _BK, _BK), :],
                bufs.at[pl.ds(slot * _BK, _BK), :],
                adj_sems.at[slot]).start()

        h = acc + b1_ref[...]
        h = jnp.where(h > 0, h, 0.01 * h)
        r = jnp.tanh(
            jnp.dot(h, w2_ref[...], preferred_element_type=jnp.float32)
            + b2_ref[...])

        if i >= 2:
            pltpu.make_async_copy(
                ostg.at[pl.ds(oslot * _BK, _BK), :],
                o_hbm.at[pl.ds((i - 2) * _BK, _BK), :],
                out_sems.at[oslot]).wait()

        ostg[pl.ds(oslot * _BK, _BK), :] = r
        pltpu.make_async_copy(
            ostg.at[pl.ds(oslot * _BK, _BK), :],
            o_hbm.at[pl.ds(i * _BK, _BK), :],
            out_sems.at[oslot]).start()

    for j in (_NB - 2, _NB - 1):
        oslot = j % 2
        pltpu.make_async_copy(
            ostg.at[pl.ds(oslot * _BK, _BK), :],
            o_hbm.at[pl.ds(j * _BK, _BK), :],
            out_sems.at[oslot]).wait()


def kernel(x, adj, W1, b1, W2, b2):
    b1r = b1.reshape(1, _HIDDEN)
    b2r = b2.reshape(1, _OUTFEAT)

    return pl.pallas_call(
        _body,
        in_specs=[
            pl.BlockSpec(memory_space=pltpu.MemorySpace.HBM),
            pl.BlockSpec(memory_space=pltpu.MemorySpace.HBM),
            pl.BlockSpec(memory_space=pltpu.MemorySpace.VMEM),
            pl.BlockSpec(memory_space=pltpu.MemorySpace.VMEM),
            pl.BlockSpec(memory_space=pltpu.MemorySpace.VMEM),
            pl.BlockSpec(memory_space=pltpu.MemorySpace.VMEM),
        ],
        out_specs=pl.BlockSpec(memory_space=pltpu.MemorySpace.HBM),
        out_shape=jax.ShapeDtypeStruct((_N, _OUTFEAT), jnp.float32),
        scratch_shapes=[
            pltpu.VMEM((_N, _INFEAT), jnp.float32),
            pltpu.VMEM((_N, _HIDDEN), jnp.float32),
            pltpu.VMEM((_NS * _BK, _N), jnp.float32),
            pltpu.VMEM((2 * _BK, _OUTFEAT), jnp.float32),
            pltpu.SemaphoreType.DMA((_NS,)),
            pltpu.SemaphoreType.DMA,
            pltpu.SemaphoreType.DMA((2,)),
        ],
        compiler_params=pltpu.CompilerParams(
            vmem_limit_bytes=64 * 1024 * 1024),
    )(adj, x, W1, b1r, W2, b2r)


# speedup vs baseline: 1.1172x; 1.0291x over previous
import jax
import jax.numpy as jnp
from jax.experimental import pallas as pl
from jax.experimental.pallas import tpu as pltpu

_N = 10000
_INFEAT = 128
_HIDDEN = 24
_OUTFEAT = 128
_BM = 400


def _body(x_ref, adj_ref, w1_ref, b1_ref, w2_ref, b2_ref, o_ref, s_ref):
    @pl.when(pl.program_id(0) == 0)
    def _():
        s_ref[...] = jnp.dot(x_ref[...], w1_ref[...],
                             preferred_element_type=jnp.float32)

    acc = jnp.dot(adj_ref[...], s_ref[...],
                  preferred_element_type=jnp.float32)
    h = acc + b1_ref[...]
    h = jnp.where(h > 0, h, 0.01 * h)
    o_ref[...] = jnp.tanh(
        jnp.dot(h, w2_ref[...], preferred_element_type=jnp.float32)
        + b2_ref[...])


def kernel(x, adj, W1, b1, W2, b2):
    b1r = b1.reshape(1, _HIDDEN)
    b2r = b2.reshape(1, _OUTFEAT)

    return pl.pallas_call(
        _body,
        grid=(_N // _BM,),
        in_specs=[
            pl.BlockSpec((_N, _INFEAT), lambda i: (0, 0)),
            pl.BlockSpec((_BM, _N), lambda i: (i, 0)),
            pl.BlockSpec((_INFEAT, _HIDDEN), lambda i: (0, 0)),
            pl.BlockSpec((1, _HIDDEN), lambda i: (0, 0)),
            pl.BlockSpec((_HIDDEN, _OUTFEAT), lambda i: (0, 0)),
            pl.BlockSpec((1, _OUTFEAT), lambda i: (0, 0)),
        ],
        out_specs=pl.BlockSpec((_BM, _OUTFEAT), lambda i: (i, 0)),
        out_shape=jax.ShapeDtypeStruct((_N, _OUTFEAT), jnp.float32),
        scratch_shapes=[pltpu.VMEM((_N, _HIDDEN), jnp.float32)],
    )(x, adj, W1, b1r, W2, b2r)
